# baseline (speedup 1.0000x reference)
import functools

import jax
import jax.numpy as jnp
import numpy as np
from jax import lax
from jax.experimental import pallas as pl
from jax.experimental.pallas import tpu as pltpu
from jax.experimental.pallas import tpu_sc as plsc

MIN_NORM = 1e-15
EPS = 1e-5

N, D, E = 10000, 128, 320000
DP = 144
NC, NS = 2, 16
NW = NC * NS
EPW = E // NW
K = 125
CHUNKS = EPW // K
NA = 10112
RPT = NA // NS
BLK = 10000


MAXNORM = 1.0 - EPS


def _artanh(x):
    x = jnp.clip(x, -1.0 + 1e-7, 1.0 - 1e-7)
    return 0.5 * jnp.log((1.0 + x) / (1.0 - x))


def _norm(x):
    return jnp.clip(jnp.sqrt(jnp.sum(x * x, axis=-1, keepdims=True)), MIN_NORM, None)


def _proj(x):
    norm = _norm(x)
    return x * jnp.minimum(MAXNORM / norm, 1.0)


def _expmap0(u):
    u_norm = _norm(u)
    return u * (jnp.tanh(u_norm) / u_norm)


def _logmap0(p):
    p_norm = _norm(p)
    return p * (_artanh(p_norm) / p_norm)


def _mobius_add(x, y):
    x2 = jnp.sum(x * x, axis=-1, keepdims=True)
    y2 = jnp.sum(y * y, axis=-1, keepdims=True)
    xy = jnp.sum(x * y, axis=-1, keepdims=True)
    rden = 1.0 / jnp.clip(1.0 + 2.0 * xy + x2 * y2, MIN_NORM, None)
    return ((1.0 + 2.0 * xy + y2) * rden) * x + ((1.0 - x2) * rden) * y


def _expmap0_proj(u, u_norm):
    hn = jnp.minimum(jnp.tanh(u_norm), MAXNORM)
    return u * (hn / u_norm), jnp.clip(hn, MIN_NORM, None)


def _logmap0_proj(z):
    n = _norm(z)
    return z * (_artanh(jnp.minimum(n, MAXNORM)) / n)


def _hyplinear_logmap(h, wt, hb, x_norm, art_x):
    mx = jnp.dot(h, wt, preferred_element_type=jnp.float32)
    mx_norm = _norm(mx)
    t = jnp.tanh(mx_norm / x_norm * art_x)
    res = mx * (t / mx_norm)
    zero_mask = jnp.max(jnp.abs(mx), axis=-1, keepdims=True) == 0.0
    res = jnp.where(zero_mask, 0.0, res)
    res = res * jnp.minimum(MAXNORM / jnp.clip(t, MIN_NORM, None), 1.0)
    x2 = jnp.minimum(t, MAXNORM) ** 2
    y2 = jnp.sum(hb * hb, axis=-1, keepdims=True)
    xy = jnp.sum(res * hb, axis=-1, keepdims=True)
    rden = 1.0 / jnp.clip(1.0 + 2.0 * xy + x2 * y2, MIN_NORM, None)
    z = ((1.0 + 2.0 * xy + y2) * rden) * res + ((1.0 - x2) * rden) * hb
    return _logmap0_proj(z)


def _deg_of(acc144):
    lane = lax.broadcasted_iota(jnp.int32, acc144.shape, 1)
    deg = jnp.sum(jnp.where(lane == D, acc144, 0.0), axis=-1, keepdims=True)
    return jnp.clip(deg, 1.0, None)


_R = float(np.arctanh(1.0 - 1e-5))


def _post_agg(support):
    s_norm = _norm(support)
    ht = jnp.maximum(support * jnp.minimum(_R / s_norm, 1.0), 0.0)
    ht_norm = _norm(ht)
    h, hn = _expmap0_proj(ht, ht_norm)
    return h, hn, jnp.minimum(ht_norm, _R)


def _pad_deg_col(xt):
    lane16 = lax.broadcasted_iota(jnp.int32, (xt.shape[0], DP - D), 1)
    pad = jnp.where(lane16 == 0, 1.0, 0.0)
    return jnp.concatenate([xt, pad], axis=1)


def _pre1_body(x_ref, wt_ref, hb_ref, out_ref):
    x = x_ref[...]
    x_norm = _norm(x)
    h, hn = _expmap0_proj(x, x_norm)
    art_h = jnp.minimum(x_norm, _R)
    xt = _hyplinear_logmap(h, wt_ref[...], hb_ref[...][:1, :], hn, art_h)
    out_ref[...] = _pad_deg_col(xt)


def _pre1(x, wt, hb):
    return pl.pallas_call(
        _pre1_body,
        grid=(N // BLK,),
        in_specs=[
            pl.BlockSpec((BLK, D), lambda i: (i, 0)),
            pl.BlockSpec((D, D), lambda i: (0, 0)),
            pl.BlockSpec((8, D), lambda i: (0, 0)),
        ],
        out_specs=pl.BlockSpec((BLK, DP), lambda i: (i, 0)),
        out_shape=jax.ShapeDtypeStruct((N, DP), jnp.float32),
    )(x, wt, hb)


def _mid_body(acc_ref, wt_ref, hb_ref, emb_ref, out_ref, deg_ref):
    acc = acc_ref[0] + acc_ref[1]
    deg = _deg_of(acc)
    rdeg = 1.0 / deg
    h1, hn1, art1 = _post_agg(acc[:, :D] * rdeg)
    emb_ref[...] = h1[None]
    deg_ref[...] = jnp.broadcast_to(rdeg, (BLK, 8))
    out_ref[...] = _hyplinear_logmap(h1, wt_ref[...], hb_ref[...][:1, :], hn1, art1)


def _mid(accs, wt, hb):
    return pl.pallas_call(
        _mid_body,
        grid=(N // BLK,),
        in_specs=[
            pl.BlockSpec((NC, BLK, DP), lambda i: (0, i, 0)),
            pl.BlockSpec((D, D), lambda i: (0, 0)),
            pl.BlockSpec((8, D), lambda i: (0, 0)),
        ],
        out_specs=[
            pl.BlockSpec((1, BLK, D), lambda i: (0, i, 0)),
            pl.BlockSpec((BLK, D), lambda i: (i, 0)),
            pl.BlockSpec((BLK, 8), lambda i: (i, 0)),
        ],
        out_shape=[
            jax.ShapeDtypeStruct((2, N, D), jnp.float32),
            jax.ShapeDtypeStruct((N, D), jnp.float32),
            jax.ShapeDtypeStruct((N, 8), jnp.float32),
        ],
    )(accs, wt, hb)


def _final_body(acc2_ref, rdeg_ref, emb_in_ref, emb_ref):
    del emb_in_ref
    rdeg = rdeg_ref[...][:, :1]
    support = (acc2_ref[0] + acc2_ref[1]) * rdeg
    h2, _, _ = _post_agg(support)
    emb_ref[...] = h2[None]


def _final(accs2, rdeg, emb):
    return pl.pallas_call(
        _final_body,
        grid=(N // BLK,),
        in_specs=[
            pl.BlockSpec((NC, BLK, D), lambda i: (0, i, 0)),
            pl.BlockSpec((BLK, 8), lambda i: (i, 0)),
            pl.BlockSpec(memory_space=pl.ANY),
        ],
        out_specs=pl.BlockSpec((1, BLK, D), lambda i: (1, i, 0)),
        out_shape=jax.ShapeDtypeStruct((2, N, D), jnp.float32),
        input_output_aliases={2: 0},
    )(accs2, rdeg, emb)


def _make_sc_agg(dp, k, nbuf, tc_tiling=False):
    chunks = EPW // k

    def body1(xt_hbm, src_hbm, dst_hbm, zeros_hbm, out_hbm,
              acc, src_v, dst_v, buf0, sem0):
        c = lax.axis_index("c")
        s = lax.axis_index("s")
        wid = s * NC + c
        pltpu.sync_copy(src_hbm.at[wid], src_v)
        pltpu.sync_copy(dst_hbm.at[wid], dst_v)
        row0 = s * RPT
        pltpu.sync_copy(zeros_hbm, acc.at[pl.ds(row0, RPT)])
        plsc.subcore_barrier()

        @pl.loop(0, chunks)
        def _chunk(j):
            pltpu.async_copy(xt_hbm.at[src_v.at[j]], buf0, sem0).wait()
            pltpu.sync_copy(buf0, acc.at[dst_v.at[j]], add=True)

        plsc.subcore_barrier()
        sl = pl.ds(row0, RPT)
        pltpu.sync_copy(acc.at[sl], out_hbm.at[c, sl])

    nphase = 2
    pchunks = chunks // nphase

    def body2(xt_hbm, src_hbm, dst_hbm, zeros_hbm, out_hbm,
              acc, src_v, dst_v, buf0, buf1, sem0, sem1):
        c = lax.axis_index("c")
        s = lax.axis_index("s")
        wid = s * NC + c
        row0 = s * RPT
        pltpu.sync_copy(zeros_hbm, acc.at[pl.ds(row0, RPT)])
        plsc.subcore_barrier()

        for p in range(nphase):
            psl = pl.ds(p * pchunks, pchunks)
            pltpu.sync_copy(src_hbm.at[wid, psl], src_v)
            pltpu.sync_copy(dst_hbm.at[wid, psl], dst_v)
            pltpu.async_copy(xt_hbm.at[src_v.at[0]], buf0, sem0)

            @pl.loop(0, pchunks // 2)
            def _chunk(t):
                j0 = 2 * t
                j1 = j0 + 1
                pltpu.make_async_copy(xt_hbm.at[src_v.at[j0]], buf0, sem0).wait()
                pltpu.async_copy(xt_hbm.at[src_v.at[j1]], buf1, sem1)
                pltpu.sync_copy(buf0, acc.at[dst_v.at[j0]], add=True)
                pltpu.make_async_copy(xt_hbm.at[src_v.at[j1]], buf1, sem1).wait()
                nxt = lax.rem(j0 + 2, pchunks)
                pltpu.async_copy(xt_hbm.at[src_v.at[nxt]], buf0, sem0)
                pltpu.sync_copy(buf1, acc.at[dst_v.at[j1]], add=True)

            pltpu.make_async_copy(xt_hbm.at[src_v.at[0]], buf0, sem0).wait()

        plsc.subcore_barrier()
        sl = pl.ds(row0, RPT)
        pltpu.sync_copy(acc.at[sl], out_hbm.at[c, sl])

    if nbuf == 1:
        scratch = [
            pltpu.VMEM_SHARED((NA, dp), jnp.float32),
            pltpu.VMEM((chunks, k), jnp.int32),
            pltpu.VMEM((chunks, k), jnp.int32),
            pltpu.VMEM((k, dp), jnp.float32),
            pltpu.SemaphoreType.DMA,
        ]
    else:
        scratch = [
            pltpu.VMEM_SHARED((NA, dp), jnp.float32),
            pltpu.VMEM((pchunks, k), jnp.int32),
            pltpu.VMEM((pchunks, k), jnp.int32),
            pltpu.VMEM((k, dp), jnp.float32),
            pltpu.VMEM((k, dp), jnp.float32),
            pltpu.SemaphoreType.DMA,
            pltpu.SemaphoreType.DMA,
        ]

    return pl.kernel(
        body1 if nbuf == 1 else body2,
        out_type=jax.ShapeDtypeStruct((NC, NA, dp), jnp.float32),
        mesh=plsc.VectorSubcoreMesh(core_axis_name="c", subcore_axis_name="s"),
        compiler_params=pltpu.CompilerParams(use_tc_tiling_on_sc=tc_tiling),
        scratch_types=scratch,
    )


K144, K128 = 100, 125
_sc_agg144 = _make_sc_agg(DP, K144, 2)
_sc_agg128 = _make_sc_agg(D, K128, 2, tc_tiling=True)


def _hyp_bias(b):
    hb = _proj(_expmap0(b.reshape(1, -1)))
    return jnp.broadcast_to(hb, (8, D))


def kernel(x, edge_index, W0, b0, W1, b1):
    src3a = edge_index[0].reshape(NW, EPW // K144, K144)
    dst3a = edge_index[1].reshape(NW, EPW // K144, K144)
    src3b = edge_index[0].reshape(NW, EPW // K128, K128)
    dst3b = edge_index[1].reshape(NW, EPW // K128, K128)
    zeros144 = jnp.zeros((RPT, DP), jnp.float32)
    zeros128 = jnp.zeros((RPT, D), jnp.float32)
    hb0 = _hyp_bias(b0)
    hb1 = _hyp_bias(b1)

    xtp1 = _pre1(x, W0.T, hb0)
    accs1 = _sc_agg144(xtp1, src3a, dst3a, zeros144)
    emb0, xt2, rdeg = _mid(accs1, W1.T, hb1)
    accs2 = _sc_agg128(xt2, src3b, dst3b, zeros128)
    return _final(accs2, rdeg, emb0)


# speedup vs baseline: 1.0326x; 1.0326x over previous
import functools

import jax
import jax.numpy as jnp
import numpy as np
from jax import lax
from jax.experimental import pallas as pl
from jax.experimental.pallas import tpu as pltpu
from jax.experimental.pallas import tpu_sc as plsc

MIN_NORM = 1e-15
EPS = 1e-5

N, D, E = 10000, 128, 320000
DP = 144
NC, NS = 2, 16
NW = NC * NS
EPW = E // NW
K = 125
CHUNKS = EPW // K
NA = 10112
RPT = NA // NS
BLK = 2000


MAXNORM = 1.0 - EPS


def _artanh(x):
    x = jnp.clip(x, -1.0 + 1e-7, 1.0 - 1e-7)
    return 0.5 * jnp.log((1.0 + x) / (1.0 - x))


def _norm(x):
    return jnp.clip(jnp.sqrt(jnp.sum(x * x, axis=-1, keepdims=True)), MIN_NORM, None)


def _proj(x):
    norm = _norm(x)
    return x * jnp.minimum(MAXNORM / norm, 1.0)


def _expmap0(u):
    u_norm = _norm(u)
    return u * (jnp.tanh(u_norm) / u_norm)


def _logmap0(p):
    p_norm = _norm(p)
    return p * (_artanh(p_norm) / p_norm)


def _mobius_add(x, y):
    x2 = jnp.sum(x * x, axis=-1, keepdims=True)
    y2 = jnp.sum(y * y, axis=-1, keepdims=True)
    xy = jnp.sum(x * y, axis=-1, keepdims=True)
    rden = 1.0 / jnp.clip(1.0 + 2.0 * xy + x2 * y2, MIN_NORM, None)
    return ((1.0 + 2.0 * xy + y2) * rden) * x + ((1.0 - x2) * rden) * y


def _expmap0_proj(u, u_norm):
    hn = jnp.minimum(jnp.tanh(u_norm), MAXNORM)
    return u * (hn / u_norm), jnp.clip(hn, MIN_NORM, None)


def _logmap0_proj(z):
    n = _norm(z)
    return z * (_artanh(jnp.minimum(n, MAXNORM)) / n)


def _hyplinear_logmap(h, wt, hb, x_norm, art_x):
    mx = jnp.dot(h, wt, preferred_element_type=jnp.float32)
    mx_norm = _norm(mx)
    t = jnp.tanh(mx_norm / x_norm * art_x)
    res = mx * (t / mx_norm)
    zero_mask = jnp.max(jnp.abs(mx), axis=-1, keepdims=True) == 0.0
    res = jnp.where(zero_mask, 0.0, res)
    res = res * jnp.minimum(MAXNORM / jnp.clip(t, MIN_NORM, None), 1.0)
    x2 = jnp.minimum(t, MAXNORM) ** 2
    y2 = jnp.sum(hb * hb, axis=-1, keepdims=True)
    xy = jnp.sum(res * hb, axis=-1, keepdims=True)
    rden = 1.0 / jnp.clip(1.0 + 2.0 * xy + x2 * y2, MIN_NORM, None)
    z = ((1.0 + 2.0 * xy + y2) * rden) * res + ((1.0 - x2) * rden) * hb
    return _logmap0_proj(z)


def _deg_of(acc144):
    lane = lax.broadcasted_iota(jnp.int32, acc144.shape, 1)
    deg = jnp.sum(jnp.where(lane == D, acc144, 0.0), axis=-1, keepdims=True)
    return jnp.clip(deg, 1.0, None)


_R = float(np.arctanh(1.0 - 1e-5))


def _post_agg(support):
    s_norm = _norm(support)
    ht = jnp.maximum(support * jnp.minimum(_R / s_norm, 1.0), 0.0)
    ht_norm = _norm(ht)
    h, hn = _expmap0_proj(ht, ht_norm)
    return h, hn, jnp.minimum(ht_norm, _R)


def _pad_deg_col(xt):
    lane16 = lax.broadcasted_iota(jnp.int32, (xt.shape[0], DP - D), 1)
    pad = jnp.where(lane16 == 0, 1.0, 0.0)
    return jnp.concatenate([xt, pad], axis=1)


def _pre1_body(x_ref, wt_ref, hb_ref, out_ref):
    x = x_ref[...]
    x_norm = _norm(x)
    h, hn = _expmap0_proj(x, x_norm)
    art_h = jnp.minimum(x_norm, _R)
    xt = _hyplinear_logmap(h, wt_ref[...], hb_ref[...][:1, :], hn, art_h)
    out_ref[...] = _pad_deg_col(xt)


def _pre1(x, wt, hb):
    return pl.pallas_call(
        _pre1_body,
        grid=(N // BLK,),
        in_specs=[
            pl.BlockSpec((BLK, D), lambda i: (i, 0)),
            pl.BlockSpec((D, D), lambda i: (0, 0)),
            pl.BlockSpec((8, D), lambda i: (0, 0)),
        ],
        out_specs=pl.BlockSpec((BLK, DP), lambda i: (i, 0)),
        out_shape=jax.ShapeDtypeStruct((N, DP), jnp.float32),
    )(x, wt, hb)


def _mid_body(acc_ref, wt_ref, hb_ref, emb_ref, out_ref, deg_ref):
    acc = acc_ref[0] + acc_ref[1]
    deg = _deg_of(acc)
    rdeg = 1.0 / deg
    h1, hn1, art1 = _post_agg(acc[:, :D] * rdeg)
    emb_ref[...] = h1[None]
    deg_ref[...] = jnp.broadcast_to(rdeg, (BLK, 8))
    out_ref[...] = _hyplinear_logmap(h1, wt_ref[...], hb_ref[...][:1, :], hn1, art1)


def _mid(accs, wt, hb):
    return pl.pallas_call(
        _mid_body,
        grid=(N // BLK,),
        in_specs=[
            pl.BlockSpec((NC, BLK, DP), lambda i: (0, i, 0)),
            pl.BlockSpec((D, D), lambda i: (0, 0)),
            pl.BlockSpec((8, D), lambda i: (0, 0)),
        ],
        out_specs=[
            pl.BlockSpec((1, BLK, D), lambda i: (0, i, 0)),
            pl.BlockSpec((BLK, D), lambda i: (i, 0)),
            pl.BlockSpec((BLK, 8), lambda i: (i, 0)),
        ],
        out_shape=[
            jax.ShapeDtypeStruct((2, N, D), jnp.float32),
            jax.ShapeDtypeStruct((N, D), jnp.float32),
            jax.ShapeDtypeStruct((N, 8), jnp.float32),
        ],
    )(accs, wt, hb)


def _final_body(acc2_ref, rdeg_ref, emb_in_ref, emb_ref):
    del emb_in_ref
    rdeg = rdeg_ref[...][:, :1]
    support = (acc2_ref[0] + acc2_ref[1]) * rdeg
    h2, _, _ = _post_agg(support)
    emb_ref[...] = h2[None]


def _final(accs2, rdeg, emb):
    return pl.pallas_call(
        _final_body,
        grid=(N // BLK,),
        in_specs=[
            pl.BlockSpec((NC, BLK, D), lambda i: (0, i, 0)),
            pl.BlockSpec((BLK, 8), lambda i: (i, 0)),
            pl.BlockSpec(memory_space=pl.ANY),
        ],
        out_specs=pl.BlockSpec((1, BLK, D), lambda i: (1, i, 0)),
        out_shape=jax.ShapeDtypeStruct((2, N, D), jnp.float32),
        input_output_aliases={2: 0},
    )(accs2, rdeg, emb)


def _make_sc_agg(dp, k, nbuf, tc_tiling=False):
    chunks = EPW // k

    def body1(xt_hbm, src_hbm, dst_hbm, zeros_hbm, out_hbm,
              acc, src_v, dst_v, buf0, sem0):
        c = lax.axis_index("c")
        s = lax.axis_index("s")
        wid = s * NC + c
        pltpu.sync_copy(src_hbm.at[wid], src_v)
        pltpu.sync_copy(dst_hbm.at[wid], dst_v)
        row0 = s * RPT
        pltpu.sync_copy(zeros_hbm, acc.at[pl.ds(row0, RPT)])
        plsc.subcore_barrier()

        @pl.loop(0, chunks)
        def _chunk(j):
            pltpu.async_copy(xt_hbm.at[src_v.at[j]], buf0, sem0).wait()
            pltpu.sync_copy(buf0, acc.at[dst_v.at[j]], add=True)

        plsc.subcore_barrier()
        sl = pl.ds(row0, RPT)
        pltpu.sync_copy(acc.at[sl], out_hbm.at[c, sl])

    nphase = 2
    pchunks = chunks // nphase

    def body2(xt_hbm, src_hbm, dst_hbm, zeros_hbm, out_hbm,
              acc, src_v, dst_v, buf0, buf1, sem0, sem1):
        c = lax.axis_index("c")
        s = lax.axis_index("s")
        wid = s * NC + c
        row0 = s * RPT
        pltpu.sync_copy(zeros_hbm, acc.at[pl.ds(row0, RPT)])
        plsc.subcore_barrier()

        for p in range(nphase):
            psl = pl.ds(p * pchunks, pchunks)
            pltpu.sync_copy(src_hbm.at[wid, psl], src_v)
            pltpu.sync_copy(dst_hbm.at[wid, psl], dst_v)
            pltpu.async_copy(xt_hbm.at[src_v.at[0]], buf0, sem0)

            @pl.loop(0, pchunks // 2)
            def _chunk(t):
                j0 = 2 * t
                j1 = j0 + 1
                pltpu.make_async_copy(xt_hbm.at[src_v.at[j0]], buf0, sem0).wait()
                pltpu.async_copy(xt_hbm.at[src_v.at[j1]], buf1, sem1)
                pltpu.sync_copy(buf0, acc.at[dst_v.at[j0]], add=True)
                pltpu.make_async_copy(xt_hbm.at[src_v.at[j1]], buf1, sem1).wait()
                nxt = lax.rem(j0 + 2, pchunks)
                pltpu.async_copy(xt_hbm.at[src_v.at[nxt]], buf0, sem0)
                pltpu.sync_copy(buf1, acc.at[dst_v.at[j1]], add=True)

            pltpu.make_async_copy(xt_hbm.at[src_v.at[0]], buf0, sem0).wait()

        plsc.subcore_barrier()
        sl = pl.ds(row0, RPT)
        pltpu.sync_copy(acc.at[sl], out_hbm.at[c, sl])

    if nbuf == 1:
        scratch = [
            pltpu.VMEM_SHARED((NA, dp), jnp.float32),
            pltpu.VMEM((chunks, k), jnp.int32),
            pltpu.VMEM((chunks, k), jnp.int32),
            pltpu.VMEM((k, dp), jnp.float32),
            pltpu.SemaphoreType.DMA,
        ]
    else:
        scratch = [
            pltpu.VMEM_SHARED((NA, dp), jnp.float32),
            pltpu.VMEM((pchunks, k), jnp.int32),
            pltpu.VMEM((pchunks, k), jnp.int32),
            pltpu.VMEM((k, dp), jnp.float32),
            pltpu.VMEM((k, dp), jnp.float32),
            pltpu.SemaphoreType.DMA,
            pltpu.SemaphoreType.DMA,
        ]

    return pl.kernel(
        body1 if nbuf == 1 else body2,
        out_type=jax.ShapeDtypeStruct((NC, NA, dp), jnp.float32),
        mesh=plsc.VectorSubcoreMesh(core_axis_name="c", subcore_axis_name="s"),
        compiler_params=pltpu.CompilerParams(use_tc_tiling_on_sc=tc_tiling),
        scratch_types=scratch,
    )


K144, K128 = 100, 125
_sc_agg144 = _make_sc_agg(DP, K144, 2)
_sc_agg128 = _make_sc_agg(D, K128, 2, tc_tiling=True)


def _hyp_bias(b):
    hb = _proj(_expmap0(b.reshape(1, -1)))
    return jnp.broadcast_to(hb, (8, D))


def kernel(x, edge_index, W0, b0, W1, b1):
    src3a = edge_index[0].reshape(NW, EPW // K144, K144)
    dst3a = edge_index[1].reshape(NW, EPW // K144, K144)
    src3b = edge_index[0].reshape(NW, EPW // K128, K128)
    dst3b = edge_index[1].reshape(NW, EPW // K128, K128)
    zeros144 = jnp.zeros((RPT, DP), jnp.float32)
    zeros128 = jnp.zeros((RPT, D), jnp.float32)
    hb0 = _hyp_bias(b0)
    hb1 = _hyp_bias(b1)

    xtp1 = _pre1(x, W0.T, hb0)
    accs1 = _sc_agg144(xtp1, src3a, dst3a, zeros144)
    emb0, xt2, rdeg = _mid(accs1, W1.T, hb1)
    accs2 = _sc_agg128(xt2, src3b, dst3b, zeros128)
    return _final(accs2, rdeg, emb0)
